# baseline (speedup 1.0000x reference)
import functools

import jax
import jax.numpy as jnp
from jax import lax
from jax.experimental import pallas as pl
from jax.experimental.pallas import tpu as pltpu
from jax.experimental.pallas import tpu_sc as plsc

_EMB = 64
_NF = 4
_SPLIT = _EMB // _NF
_N_LAYERS = 2
_N_ITERS = 2

_NSC = 2
_NT = 16

_MSG_C = 256
_ATT_C = 512
_DEG_C = 1024


def _mds(start, size, mult):
    return pl.ds(pl.multiple_of(start, mult), size)


def _pad_to(x, n, axis=0):
    pad = [(0, 0)] * x.ndim
    pad[axis] = (0, n - x.shape[axis])
    return jnp.pad(x, pad)


def _mean3_body(a_ref, b_ref, c_ref, o_ref):
    o_ref[...] = (a_ref[...] + b_ref[...] + c_ref[...]) * (1.0 / 3.0)


def _mean3(a, b, c):
    n = a.shape[1]
    blk = n // 16
    return pl.pallas_call(
        _mean3_body,
        out_shape=jax.ShapeDtypeStruct(a.shape, a.dtype),
        grid=(_NSC, 16),
        in_specs=[pl.BlockSpec((1, blk, 32), lambda i, j: (i, j, 0))] * 3,
        out_specs=pl.BlockSpec((1, blk, 32), lambda i, j: (i, j, 0)),
    )(a, b, c)


def _norm_f(x):
    n = x.shape[1]
    xr = x.reshape(_NSC, n, 2, _SPLIT)
    nrm = jnp.sqrt(jnp.sum(xr * xr, axis=3, keepdims=True))
    return (xr / jnp.maximum(nrm, 1e-12)).reshape(_NSC, n, 32)


def _make_mesh():
    return plsc.VectorSubcoreMesh(core_axis_name="c", subcore_axis_name="s")


_SC_PARAMS = pltpu.CompilerParams(use_tc_tiling_on_sc=False)


def _make_deg_kernel(n_pad, e_pad):
    rows_pt = n_pad // _NT
    edges_pt = e_pad // _NT
    chunks = edges_pt // _DEG_C

    @functools.partial(
        pl.kernel,
        out_type=jax.ShapeDtypeStruct((_NSC, n_pad, 2), jnp.float32),
        mesh=_make_mesh(),
        compiler_params=_SC_PARAMS,
        scratch_types=[
            pltpu.VMEM_SHARED((n_pad, 2), jnp.float32),
            pltpu.VMEM((_DEG_C // 128, 128), jnp.int32),
            pltpu.VMEM((_DEG_C, 2), jnp.float32),
            pltpu.SemaphoreType.DMA,
        ],
    )
    def deg_kernel(sv, h2, z2, dout, acc, idx_h, vals, sem):
        c = lax.axis_index("c")
        s = lax.axis_index("s")
        sl = _mds(s * rows_pt, rows_pt, 8)
        pltpu.sync_copy(z2.at[sl], acc.at[sl])
        plsc.subcore_barrier()
        base = s * edges_pt

        def chunk(i, carry):
            eb = base + i * _DEG_C
            pltpu.sync_copy(h2.at[_mds(eb // 128, _DEG_C // 128, 8)], idx_h)
            pltpu.async_copy(sv.at[c].at[_mds(eb, _DEG_C, 128)], vals,
                             sem).wait()
            for j in range(_DEG_C // 128):
                pltpu.sync_copy(vals.at[pl.ds(j * 128, 128)],
                                acc.at[idx_h.at[j]], add=True)
            return carry

        lax.fori_loop(0, chunks, chunk, 0)
        plsc.subcore_barrier()
        pltpu.sync_copy(acc.at[sl], dout.at[c, sl])

    return deg_kernel


def _make_msg_kernel(n_pad, e_pad):
    rows_pt = n_pad // _NT
    edges_pt = e_pad // _NT
    chunks = edges_pt // _MSG_C

    @functools.partial(
        pl.kernel,
        out_type=jax.ShapeDtypeStruct((_NSC, n_pad, 32), jnp.float32),
        mesh=_make_mesh(),
        compiler_params=_SC_PARAMS,
        scratch_types=[
            pltpu.VMEM_SHARED((n_pad, 32), jnp.float32),
            pltpu.VMEM((_MSG_C // 128, 128), jnp.int32),
            pltpu.VMEM((_MSG_C // 128, 128), jnp.int32),
            pltpu.VMEM((_MSG_C, 32), jnp.float32),
            pltpu.VMEM((_MSG_C * 2,), jnp.float32),
            pltpu.SemaphoreType.DMA,
        ],
    )
    def msg_kernel(y3, sv, h2, t2, z32, fe3,
                   acc, idx_h, idx_t, rows, sbuf, sem):
        c = lax.axis_index("c")
        s = lax.axis_index("s")
        sl = _mds(s * rows_pt, rows_pt, 8)
        pltpu.sync_copy(z32.at[sl], acc.at[sl])
        plsc.subcore_barrier()
        base = s * edges_pt

        def chunk(i, carry):
            eb = base + i * _MSG_C
            rb = eb // 128
            pltpu.sync_copy(t2.at[_mds(rb, _MSG_C // 128, 2)], idx_t)
            pltpu.sync_copy(h2.at[_mds(rb, _MSG_C // 128, 2)], idx_h)
            cps = [pltpu.async_copy(sv.at[c].at[_mds(eb * 2, _MSG_C * 2, 256)],
                                    sbuf, sem)]
            for j in range(_MSG_C // 128):
                cps.append(pltpu.async_copy(
                    y3.at[c].at[idx_t.at[j]],
                    rows.at[pl.ds(j * 128, 128)], sem))
            for cp in cps:
                cp.wait()

            def mulgrp(g, cr):
                v = sbuf[_mds(g * 16, 16, 16)]
                for k in range(8):
                    r = g * 8 + k
                    rows[r, pl.ds(0, 16)] = rows[r, pl.ds(0, 16)] * v[2 * k]
                    rows[r, pl.ds(16, 16)] = rows[r, pl.ds(16, 16)] * v[2 * k + 1]
                return cr

            lax.fori_loop(0, _MSG_C // 8, mulgrp, 0)
            for j in range(_MSG_C // 128):
                pltpu.sync_copy(rows.at[pl.ds(j * 128, 128)],
                                acc.at[idx_h.at[j]], add=True)
            return carry

        lax.fori_loop(0, chunks, chunk, 0)
        plsc.subcore_barrier()
        pltpu.sync_copy(acc.at[sl], fe3.at[c, sl])

    return msg_kernel


def _make_att_kernel(n_pad, e_pad):
    edges_pt = e_pad // _NT
    chunks = edges_pt // _ATT_C

    @functools.partial(
        pl.kernel,
        out_type=jax.ShapeDtypeStruct((_NSC, e_pad, 32), jnp.float32),
        mesh=_make_mesh(),
        compiler_params=_SC_PARAMS,
        scratch_types=[
            pltpu.VMEM((_ATT_C // 128, 128), jnp.int32),
            pltpu.VMEM((_ATT_C // 128, 128), jnp.int32),
            pltpu.VMEM((_ATT_C, 32), jnp.float32),
            pltpu.VMEM((_ATT_C, 32), jnp.float32),
            pltpu.SemaphoreType.DMA,
        ],
    )
    def att_kernel(fn, tn, h2, t2, out, idx_h, idx_t, ra, rb, sem):
        c = lax.axis_index("c")
        s = lax.axis_index("s")
        base = s * edges_pt

        def chunk(i, carry):
            eb = base + i * _ATT_C
            rbase = eb // 128
            pltpu.sync_copy(h2.at[_mds(rbase, _ATT_C // 128, 4)], idx_h)
            pltpu.sync_copy(t2.at[_mds(rbase, _ATT_C // 128, 4)], idx_t)
            cps = []
            for j in range(_ATT_C // 128):
                cps.append(pltpu.async_copy(
                    fn.at[c].at[idx_h.at[j]], ra.at[pl.ds(j * 128, 128)],
                    sem))
                cps.append(pltpu.async_copy(
                    tn.at[c].at[idx_t.at[j]], rb.at[pl.ds(j * 128, 128)],
                    sem))
            for cp in cps:
                cp.wait()

            def mulrow(r, cr):
                rows0 = ra[r, pl.ds(0, 16)] * rb[r, pl.ds(0, 16)]
                rows1 = ra[r, pl.ds(16, 16)] * rb[r, pl.ds(16, 16)]
                ra[r, pl.ds(0, 16)] = rows0
                ra[r, pl.ds(16, 16)] = rows1
                return cr

            lax.fori_loop(0, _ATT_C, mulrow, 0)
            pltpu.sync_copy(ra, out.at[c].at[_mds(eb, _ATT_C, 128)])
            return carry

        lax.fori_loop(0, chunks, chunk, 0)

    return att_kernel


def kernel(user_embedding, item_embedding, all_h, all_t):
    n_users = user_embedding.shape[0]
    N = n_users + item_embedding.shape[0]
    E = all_h.shape[0]
    n_pad = ((N + 1 + _NT * 8 - 1) // (_NT * 8)) * (_NT * 8)
    epc = _NT * _DEG_C
    e_pad = ((E + epc - 1) // epc) * epc

    deg_k = _make_deg_kernel(n_pad, e_pad)
    msg_k = _make_msg_kernel(n_pad, e_pad)
    att_k = _make_att_kernel(n_pad, e_pad)

    ego = jnp.concatenate([user_embedding, item_embedding], axis=0)
    ego = _pad_to(ego, n_pad)
    ego_s = ego.reshape(n_pad, _NSC, 32).transpose(1, 0, 2)
    h_p = _pad_to(all_h, e_pad).at[E:].set(N)
    t_p = _pad_to(all_t, e_pad).at[E:].set(N)
    h2 = h_p.reshape(e_pad // 128, 128)
    t2 = t_p.reshape(e_pad // 128, 128)
    z2 = jnp.zeros((n_pad, 2), jnp.float32)
    z32 = jnp.zeros((n_pad, 32), jnp.float32)

    A = jnp.ones((_NSC, e_pad, 2), dtype=jnp.float32)
    layer_outs = [ego_s]
    for layer in range(_N_LAYERS):
        Tn = jnp.tanh(_norm_f(ego_s))
        ego_r = ego_s.reshape(_NSC, n_pad, 2, _SPLIT)
        fe = None
        for it in range(_N_ITERS):
            am = jnp.max(A, axis=(0, 2), keepdims=True)
            ex = jnp.exp(A - am)
            scores = ex / jnp.sum(ex, axis=(0, 2), keepdims=True)
            deg = deg_k(scores, h2, z2)
            d_col = lax.rsqrt(jnp.maximum(deg, 1e-30))
            y_s = (ego_r * d_col[:, :, :, None]).reshape(_NSC, n_pad, 32)
            fe3 = msg_k(y_s, scores.reshape(_NSC, -1), h2, t2, z32)
            fe = (fe3.reshape(_NSC, n_pad, 2, _SPLIT) * d_col[:, :, :, None]
                  ).reshape(_NSC, n_pad, 32)
            last_step = layer == _N_LAYERS - 1 and it == _N_ITERS - 1
            if not last_step:
                Fn = _norm_f(fe)
                P = att_k(Fn, Tn, h2, t2)
                A = A + jnp.sum(P.reshape(_NSC, e_pad, 2, _SPLIT), axis=3)
        ego_s = fe
        layer_outs.append(ego_s)
    all_s = _mean3(*layer_outs)
    all_emb = all_s.transpose(1, 0, 2).reshape(n_pad, _EMB)
    return all_emb[:n_users], all_emb[n_users:N]


# speedup vs baseline: 2.7860x; 2.7860x over previous
import functools

import jax
import jax.numpy as jnp
from jax import lax
from jax.experimental import pallas as pl
from jax.experimental.pallas import tpu as pltpu
from jax.experimental.pallas import tpu_sc as plsc

_EMB = 64
_NF = 4
_SPLIT = _EMB // _NF
_N_LAYERS = 2
_N_ITERS = 2

_NSC = 2
_NT = 16

_MSG_C = 512
_ATT_C = 512
_DEG_C = 1024


def _mds(start, size, mult):
    return pl.ds(pl.multiple_of(start, mult), size)


def _pad_to(x, n, axis=0):
    pad = [(0, 0)] * x.ndim
    pad[axis] = (0, n - x.shape[axis])
    return jnp.pad(x, pad)


def _mean3_body(a_ref, b_ref, c_ref, o_ref):
    o_ref[...] = (a_ref[...] + b_ref[...] + c_ref[...]) * (1.0 / 3.0)


def _mean3(a, b, c):
    n = a.shape[1]
    blk = n // 16
    return pl.pallas_call(
        _mean3_body,
        out_shape=jax.ShapeDtypeStruct(a.shape, a.dtype),
        grid=(_NSC, 16),
        in_specs=[pl.BlockSpec((1, blk, 32), lambda i, j: (i, j, 0))] * 3,
        out_specs=pl.BlockSpec((1, blk, 32), lambda i, j: (i, j, 0)),
    )(a, b, c)


def _norm_f(x):
    n = x.shape[1]
    xr = x.reshape(_NSC, n, 2, _SPLIT)
    nrm = jnp.sqrt(jnp.sum(xr * xr, axis=3, keepdims=True))
    return (xr / jnp.maximum(nrm, 1e-12)).reshape(_NSC, n, 32)


def _make_mesh():
    return plsc.VectorSubcoreMesh(core_axis_name="c", subcore_axis_name="s")


_SC_PARAMS = pltpu.CompilerParams(use_tc_tiling_on_sc=False,
                                  needs_layout_passes=False)


def _make_deg_kernel(n_pad, e_pad):
    rows_pt2 = n_pad * 2 // _NT
    edges_pt = e_pad // _NT
    chunks = edges_pt // _DEG_C
    fpc = _DEG_C * 2 // 128

    @functools.partial(
        pl.kernel,
        out_type=jax.ShapeDtypeStruct((_NSC, n_pad * 2), jnp.float32),
        mesh=_make_mesh(),
        compiler_params=_SC_PARAMS,
        scratch_types=[
            pltpu.VMEM_SHARED((n_pad * 2,), jnp.float32),
            pltpu.VMEM((fpc, 128), jnp.int32),
            pltpu.VMEM((fpc, 128), jnp.float32),
            pltpu.SemaphoreType.DMA,
        ],
    )
    def deg_kernel(sv, h2x2, z2f, dout, acc, idx2, vals, sem):
        c = lax.axis_index("c")
        s = lax.axis_index("s")
        sl = _mds(s * rows_pt2, rows_pt2, 8)
        pltpu.sync_copy(z2f.at[sl], acc.at[sl])
        plsc.subcore_barrier()
        base = s * edges_pt

        def chunk(i, carry):
            fb = (base + i * _DEG_C) // 64
            pltpu.sync_copy(h2x2.at[_mds(fb, fpc, 8)], idx2)
            pltpu.async_copy(sv.at[c].at[_mds(fb, fpc, 8)], vals, sem).wait()
            for j in range(fpc):
                pltpu.sync_copy(vals.at[j], acc.at[idx2.at[j]], add=True)
            return carry

        lax.fori_loop(0, chunks, chunk, 0)
        plsc.subcore_barrier()
        pltpu.sync_copy(acc.at[sl], dout.at[c, sl])

    return deg_kernel


def _make_msg_kernel(n_pad, e_pad):
    rows_pt = n_pad // _NT
    edges_pt = e_pad // _NT
    chunks = edges_pt // _MSG_C

    @functools.partial(
        pl.kernel,
        out_type=jax.ShapeDtypeStruct((_NSC, n_pad, 32), jnp.float32),
        mesh=_make_mesh(),
        compiler_params=_SC_PARAMS,
        scratch_types=[
            pltpu.VMEM_SHARED((n_pad, 32), jnp.float32),
            pltpu.VMEM((_MSG_C // 128, 128), jnp.int32),
            pltpu.VMEM((_MSG_C // 128, 128), jnp.int32),
            pltpu.VMEM((_MSG_C, 32), jnp.float32),
            pltpu.VMEM((_MSG_C * 2 // 128, 128), jnp.float32),
            pltpu.SemaphoreType.DMA,
        ],
    )
    def msg_kernel(y3, sv, h2, t2, z32, fe3,
                   acc, idx_h, idx_t, rows, sbuf, sem):
        c = lax.axis_index("c")
        s = lax.axis_index("s")
        sl = _mds(s * rows_pt, rows_pt, 8)
        pltpu.sync_copy(z32.at[sl], acc.at[sl])
        plsc.subcore_barrier()
        base = s * edges_pt

        def chunk(i, carry):
            eb = base + i * _MSG_C
            rb = eb // 128
            pltpu.sync_copy(t2.at[_mds(rb, _MSG_C // 128, 4)], idx_t)
            pltpu.sync_copy(h2.at[_mds(rb, _MSG_C // 128, 4)], idx_h)
            cps = [pltpu.async_copy(
                sv.at[c].at[_mds(eb // 64, _MSG_C * 2 // 128, 8)], sbuf, sem)]
            for j in range(_MSG_C // 128):
                cps.append(pltpu.async_copy(
                    y3.at[c].at[idx_t.at[j]],
                    rows.at[pl.ds(j * 128, 128)], sem))
            for cp in cps:
                cp.wait()

            def mulgrp(g, cr):
                v = sbuf[g // 8, _mds((g % 8) * 16, 16, 16)]
                for k in range(8):
                    r = g * 8 + k
                    rows[r, pl.ds(0, 16)] = rows[r, pl.ds(0, 16)] * v[2 * k]
                    rows[r, pl.ds(16, 16)] = rows[r, pl.ds(16, 16)] * v[2 * k + 1]
                return cr

            lax.fori_loop(0, _MSG_C // 8, mulgrp, 0)
            for j in range(_MSG_C // 128):
                pltpu.sync_copy(rows.at[pl.ds(j * 128, 128)],
                                acc.at[idx_h.at[j]], add=True)
            return carry

        lax.fori_loop(0, chunks, chunk, 0)
        plsc.subcore_barrier()
        pltpu.sync_copy(acc.at[sl], fe3.at[c, sl])

    return msg_kernel


def _make_att_kernel(n_pad, e_pad):
    edges_pt = e_pad // _NT
    chunks = edges_pt // _ATT_C
    fpc = _ATT_C * 2 // 128

    @functools.partial(
        pl.kernel,
        out_type=jax.ShapeDtypeStruct((_NSC, e_pad * 2 // 128, 128),
                                      jnp.float32),
        mesh=_make_mesh(),
        compiler_params=_SC_PARAMS,
        scratch_types=[
            pltpu.VMEM((_ATT_C // 128, 128), jnp.int32),
            pltpu.VMEM((_ATT_C // 128, 128), jnp.int32),
            pltpu.VMEM((_ATT_C, 32), jnp.float32),
            pltpu.VMEM((_ATT_C, 32), jnp.float32),
            pltpu.VMEM((16, fpc, 128), jnp.float32),
            pltpu.VMEM((fpc, 128), jnp.float32),
            pltpu.SemaphoreType.DMA,
        ],
    )
    def att_kernel(fn, tn, h2, t2, out, idx_h, idx_t, ra, rb, pt, psum, sem):
        c = lax.axis_index("c")
        s = lax.axis_index("s")
        base = s * edges_pt
        kio = lax.iota(jnp.int32, 16)

        def chunk(i, carry):
            eb = base + i * _ATT_C
            rbase = eb // 128
            pltpu.sync_copy(h2.at[_mds(rbase, _ATT_C // 128, 4)], idx_h)
            pltpu.sync_copy(t2.at[_mds(rbase, _ATT_C // 128, 4)], idx_t)
            cps = []
            for j in range(_ATT_C // 128):
                cps.append(pltpu.async_copy(
                    fn.at[c].at[idx_h.at[j]], ra.at[pl.ds(j * 128, 128)],
                    sem))
                cps.append(pltpu.async_copy(
                    tn.at[c].at[idx_t.at[j]], rb.at[pl.ds(j * 128, 128)],
                    sem))
            for cp in cps:
                cp.wait()

            def mulrow(r, cr):
                jj = r // 64
                col = (r % 64) * 2
                v0 = ra[r, pl.ds(0, 16)] * rb[r, pl.ds(0, 16)]
                v1 = ra[r, pl.ds(16, 16)] * rb[r, pl.ds(16, 16)]
                jv = jnp.full((16,), jj, jnp.int32)
                cv = jnp.full((16,), col, jnp.int32)
                plsc.store_scatter(pt, [kio, jv, cv], v0)
                plsc.store_scatter(pt, [kio, jv, cv + 1], v1)
                return cr

            lax.fori_loop(0, _ATT_C, mulrow, 0)

            def redgrp(g, cr):
                jj = g // 8
                col = (g % 8) * 16
                acc16 = pt[0, jj, _mds(col, 16, 16)]
                for k in range(1, 16):
                    acc16 = acc16 + pt[k, jj, _mds(col, 16, 16)]
                psum[jj, _mds(col, 16, 16)] = acc16
                return cr

            lax.fori_loop(0, fpc * 8, redgrp, 0)
            pltpu.sync_copy(psum, out.at[c].at[_mds(eb // 64, fpc, 8)])
            return carry

        lax.fori_loop(0, chunks, chunk, 0)

    return att_kernel


def kernel(user_embedding, item_embedding, all_h, all_t):
    n_users = user_embedding.shape[0]
    N = n_users + item_embedding.shape[0]
    E = all_h.shape[0]
    n_pad = ((N + 1 + _NT * 64 - 1) // (_NT * 64)) * (_NT * 64)
    epc = _NT * _DEG_C
    e_pad = ((E + epc - 1) // epc) * epc
    er = e_pad * 2 // 128

    deg_k = _make_deg_kernel(n_pad, e_pad)
    msg_k = _make_msg_kernel(n_pad, e_pad)
    att_k = _make_att_kernel(n_pad, e_pad)

    ego = jnp.concatenate([user_embedding, item_embedding], axis=0)
    ego = _pad_to(ego, n_pad)
    ego_s = ego.reshape(n_pad, _NSC, 32).transpose(1, 0, 2)
    h_p = _pad_to(all_h, e_pad).at[E:].set(N)
    t_p = _pad_to(all_t, e_pad).at[E:].set(N)
    h2 = h_p.reshape(e_pad // 128, 128)
    t2 = t_p.reshape(e_pad // 128, 128)
    h2x2 = (jnp.repeat(h_p.reshape(er, 64) * 2, 2, axis=1)
            + (jnp.arange(128, dtype=jnp.int32) % 2)[None, :])
    z2f = jnp.zeros((n_pad * 2,), jnp.float32)
    z32 = jnp.zeros((n_pad, 32), jnp.float32)

    A = jnp.ones((_NSC, er, 128), dtype=jnp.float32)
    lane_even = (jnp.arange(128) % 2 == 0)
    layer_outs = [ego_s]
    for layer in range(_N_LAYERS):
        Tn = jnp.tanh(_norm_f(ego_s))
        ego_r = ego_s.reshape(_NSC, n_pad, 2, _SPLIT)
        fe = None
        for it in range(_N_ITERS):
            ex = jnp.exp(A)
            den = (ex[0] + ex[1])
            den = den[:, 0::2] + den[:, 1::2]
            scores = ex / jnp.repeat(den, 2, axis=1)[None]
            dflat = deg_k(scores, h2x2, z2f)
            d_col = lax.rsqrt(jnp.maximum(dflat, 1e-30)).reshape(
                _NSC, n_pad, 2)
            y_s = (ego_r * d_col[:, :, :, None]).reshape(_NSC, n_pad, 32)
            fe3 = msg_k(y_s, scores, h2, t2, z32)
            fe = (fe3.reshape(_NSC, n_pad, 2, _SPLIT) * d_col[:, :, :, None]
                  ).reshape(_NSC, n_pad, 32)
            last_step = layer == _N_LAYERS - 1 and it == _N_ITERS - 1
            if not last_step:
                Fn = _norm_f(fe)
                A = A + att_k(Fn, Tn, h2, t2)
        ego_s = fe
        layer_outs.append(ego_s)
    all_s = _mean3(*layer_outs)
    all_emb = all_s.transpose(1, 0, 2).reshape(n_pad, _EMB)
    return all_emb[:n_users], all_emb[n_users:N]


# speedup vs baseline: 3.0568x; 1.0972x over previous
import functools

import jax
import jax.numpy as jnp
from jax import lax
from jax.experimental import pallas as pl
from jax.experimental.pallas import tpu as pltpu
from jax.experimental.pallas import tpu_sc as plsc

_EMB = 64
_NF = 4
_SPLIT = _EMB // _NF
_N_LAYERS = 2
_N_ITERS = 2

_NSC = 2
_NT = 16

_MSG_C = 512
_ATT_C = 512
_DEG_C = 1024


def _mds(start, size, mult):
    return pl.ds(pl.multiple_of(start, mult), size)


def _pad_to(x, n, axis=0):
    pad = [(0, 0)] * x.ndim
    pad[axis] = (0, n - x.shape[axis])
    return jnp.pad(x, pad)


def _mean3_body(a_ref, b_ref, c_ref, o_ref):
    o_ref[...] = (a_ref[...] + b_ref[...] + c_ref[...]) * (1.0 / 3.0)


def _mean3(a, b, c):
    n = a.shape[1]
    blk = n // 16
    return pl.pallas_call(
        _mean3_body,
        out_shape=jax.ShapeDtypeStruct(a.shape, a.dtype),
        grid=(_NSC, 16),
        in_specs=[pl.BlockSpec((1, blk, 32), lambda i, j: (i, j, 0))] * 3,
        out_specs=pl.BlockSpec((1, blk, 32), lambda i, j: (i, j, 0)),
    )(a, b, c)


def _norm_f(x):
    n = x.shape[1]
    xr = x.reshape(_NSC, n, 2, _SPLIT)
    nrm = jnp.sqrt(jnp.sum(xr * xr, axis=3, keepdims=True))
    return (xr / jnp.maximum(nrm, 1e-12)).reshape(_NSC, n, 32)


def _make_mesh():
    return plsc.VectorSubcoreMesh(core_axis_name="c", subcore_axis_name="s")


_SC_PARAMS = pltpu.CompilerParams(use_tc_tiling_on_sc=False,
                                  needs_layout_passes=False)


def _make_deg_kernel(n_pad, e_pad):
    rows_pt2 = n_pad * 2 // _NT
    edges_pt = e_pad // _NT
    chunks = edges_pt // _DEG_C
    fpc = _DEG_C * 2 // 128

    @functools.partial(
        pl.kernel,
        out_type=jax.ShapeDtypeStruct((_NSC, n_pad * 2), jnp.float32),
        mesh=_make_mesh(),
        compiler_params=_SC_PARAMS,
        scratch_types=[
            pltpu.VMEM_SHARED((n_pad * 2,), jnp.float32),
            pltpu.VMEM((fpc, 128), jnp.int32),
            pltpu.VMEM((fpc, 128), jnp.float32),
            pltpu.SemaphoreType.DMA,
        ],
    )
    def deg_kernel(sv, h2x2, z2f, dout, acc, idx2, vals, sem):
        c = lax.axis_index("c")
        s = lax.axis_index("s")
        sl = _mds(s * rows_pt2, rows_pt2, 8)
        pltpu.sync_copy(z2f.at[sl], acc.at[sl])
        plsc.subcore_barrier()
        base = s * edges_pt

        def chunk(i, carry):
            fb = (base + i * _DEG_C) // 64
            pltpu.sync_copy(h2x2.at[_mds(fb, fpc, 8)], idx2)
            pltpu.async_copy(sv.at[c].at[_mds(fb, fpc, 8)], vals, sem).wait()
            for j in range(fpc):
                pltpu.sync_copy(vals.at[j], acc.at[idx2.at[j]], add=True)
            return carry

        lax.fori_loop(0, chunks, chunk, 0)
        plsc.subcore_barrier()
        pltpu.sync_copy(acc.at[sl], dout.at[c, sl])

    return deg_kernel


def _make_msg_kernel(n_pad, e_pad):
    rows_pt = n_pad // _NT
    edges_pt = e_pad // _NT
    chunks = edges_pt // _MSG_C

    @functools.partial(
        pl.kernel,
        out_type=jax.ShapeDtypeStruct((_NSC, n_pad, 32), jnp.float32),
        mesh=_make_mesh(),
        compiler_params=_SC_PARAMS,
        scratch_types=[
            pltpu.VMEM_SHARED((n_pad, 32), jnp.float32),
            pltpu.VMEM((_MSG_C // 128, 128), jnp.int32),
            pltpu.VMEM((_MSG_C // 128, 128), jnp.int32),
            pltpu.VMEM((_MSG_C, 32), jnp.float32),
            pltpu.VMEM((_MSG_C * 2 // 128, 128), jnp.float32),
            pltpu.SemaphoreType.DMA,
        ],
    )
    def msg_kernel(y3, sv, h2, t2, z32, fe3,
                   acc, idx_h, idx_t, rows, sbuf, sem):
        c = lax.axis_index("c")
        s = lax.axis_index("s")
        sl = _mds(s * rows_pt, rows_pt, 8)
        pltpu.sync_copy(z32.at[sl], acc.at[sl])
        plsc.subcore_barrier()
        base = s * edges_pt

        def chunk(i, carry):
            eb = base + i * _MSG_C
            rb = eb // 128
            pltpu.sync_copy(t2.at[_mds(rb, _MSG_C // 128, 4)], idx_t)
            pltpu.sync_copy(h2.at[_mds(rb, _MSG_C // 128, 4)], idx_h)
            cps = [pltpu.async_copy(
                sv.at[c].at[_mds(eb // 64, _MSG_C * 2 // 128, 8)], sbuf, sem)]
            for j in range(_MSG_C // 128):
                cps.append(pltpu.async_copy(
                    y3.at[c].at[idx_t.at[j]],
                    rows.at[pl.ds(j * 128, 128)], sem))
            for cp in cps:
                cp.wait()

            @plsc.parallel_loop(0, _MSG_C // 8, 1, unroll=4)
            def mulgrp(g):
                v = sbuf[g // 8, _mds((g % 8) * 16, 16, 16)]
                for k in range(8):
                    r = g * 8 + k
                    rows[r, pl.ds(0, 16)] = rows[r, pl.ds(0, 16)] * v[2 * k]
                    rows[r, pl.ds(16, 16)] = rows[r, pl.ds(16, 16)] * v[2 * k + 1]
            for j in range(_MSG_C // 128):
                pltpu.sync_copy(rows.at[pl.ds(j * 128, 128)],
                                acc.at[idx_h.at[j]], add=True)
            return carry

        lax.fori_loop(0, chunks, chunk, 0)
        plsc.subcore_barrier()
        pltpu.sync_copy(acc.at[sl], fe3.at[c, sl])

    return msg_kernel


def _make_att_kernel(n_pad, e_pad):
    edges_pt = e_pad // _NT
    chunks = edges_pt // _ATT_C
    fpc = _ATT_C * 2 // 128

    @functools.partial(
        pl.kernel,
        out_type=jax.ShapeDtypeStruct((_NSC, e_pad * 2 // 128, 128),
                                      jnp.float32),
        mesh=_make_mesh(),
        compiler_params=_SC_PARAMS,
        scratch_types=[
            pltpu.VMEM((_ATT_C // 128, 128), jnp.int32),
            pltpu.VMEM((_ATT_C // 128, 128), jnp.int32),
            pltpu.VMEM((_ATT_C, 32), jnp.float32),
            pltpu.VMEM((_ATT_C, 32), jnp.float32),
            pltpu.VMEM((16, fpc, 128), jnp.float32),
            pltpu.VMEM((fpc, 128), jnp.float32),
            pltpu.SemaphoreType.DMA,
        ],
    )
    def att_kernel(fn, tn, h2, t2, out, idx_h, idx_t, ra, rb, pt, psum, sem):
        c = lax.axis_index("c")
        s = lax.axis_index("s")
        base = s * edges_pt
        kio = lax.iota(jnp.int32, 16)

        def chunk(i, carry):
            eb = base + i * _ATT_C
            rbase = eb // 128
            pltpu.sync_copy(h2.at[_mds(rbase, _ATT_C // 128, 4)], idx_h)
            pltpu.sync_copy(t2.at[_mds(rbase, _ATT_C // 128, 4)], idx_t)
            cps = []
            for j in range(_ATT_C // 128):
                cps.append(pltpu.async_copy(
                    fn.at[c].at[idx_h.at[j]], ra.at[pl.ds(j * 128, 128)],
                    sem))
                cps.append(pltpu.async_copy(
                    tn.at[c].at[idx_t.at[j]], rb.at[pl.ds(j * 128, 128)],
                    sem))
            for cp in cps:
                cp.wait()

            @plsc.parallel_loop(0, _ATT_C, 1, unroll=8)
            def mulrow(r):
                jj = r // 64
                col = (r % 64) * 2
                v0 = ra[r, pl.ds(0, 16)] * rb[r, pl.ds(0, 16)]
                v1 = ra[r, pl.ds(16, 16)] * rb[r, pl.ds(16, 16)]
                jv = jnp.full((16,), jj, jnp.int32)
                cv = jnp.full((16,), col, jnp.int32)
                plsc.store_scatter(pt, [kio, jv, cv], v0)
                plsc.store_scatter(pt, [kio, jv, cv + 1], v1)

            @plsc.parallel_loop(0, fpc * 8, 1, unroll=2)
            def redgrp(g):
                jj = g // 8
                col = (g % 8) * 16
                acc16 = pt[0, jj, _mds(col, 16, 16)]
                for k in range(1, 16):
                    acc16 = acc16 + pt[k, jj, _mds(col, 16, 16)]
                psum[jj, _mds(col, 16, 16)] = acc16
            pltpu.sync_copy(psum, out.at[c].at[_mds(eb // 64, fpc, 8)])
            return carry

        lax.fori_loop(0, chunks, chunk, 0)

    return att_kernel


def kernel(user_embedding, item_embedding, all_h, all_t):
    n_users = user_embedding.shape[0]
    N = n_users + item_embedding.shape[0]
    E = all_h.shape[0]
    n_pad = ((N + 1 + _NT * 64 - 1) // (_NT * 64)) * (_NT * 64)
    epc = _NT * _DEG_C
    e_pad = ((E + epc - 1) // epc) * epc
    er = e_pad * 2 // 128

    deg_k = _make_deg_kernel(n_pad, e_pad)
    msg_k = _make_msg_kernel(n_pad, e_pad)
    att_k = _make_att_kernel(n_pad, e_pad)

    ego = jnp.concatenate([user_embedding, item_embedding], axis=0)
    ego = _pad_to(ego, n_pad)
    ego_s = ego.reshape(n_pad, _NSC, 32).transpose(1, 0, 2)
    h_p = _pad_to(all_h, e_pad).at[E:].set(N)
    t_p = _pad_to(all_t, e_pad).at[E:].set(N)
    h2 = h_p.reshape(e_pad // 128, 128)
    t2 = t_p.reshape(e_pad // 128, 128)
    h2x2 = (jnp.repeat(h_p.reshape(er, 64) * 2, 2, axis=1)
            + (jnp.arange(128, dtype=jnp.int32) % 2)[None, :])
    z2f = jnp.zeros((n_pad * 2,), jnp.float32)
    z32 = jnp.zeros((n_pad, 32), jnp.float32)

    A = jnp.ones((_NSC, er, 128), dtype=jnp.float32)
    lane_even = (jnp.arange(128) % 2 == 0)
    layer_outs = [ego_s]
    for layer in range(_N_LAYERS):
        Tn = jnp.tanh(_norm_f(ego_s))
        ego_r = ego_s.reshape(_NSC, n_pad, 2, _SPLIT)
        fe = None
        for it in range(_N_ITERS):
            ex = jnp.exp(A)
            den = (ex[0] + ex[1])
            den = den[:, 0::2] + den[:, 1::2]
            scores = ex / jnp.repeat(den, 2, axis=1)[None]
            dflat = deg_k(scores, h2x2, z2f)
            d_col = lax.rsqrt(jnp.maximum(dflat, 1e-30)).reshape(
                _NSC, n_pad, 2)
            y_s = (ego_r * d_col[:, :, :, None]).reshape(_NSC, n_pad, 32)
            fe3 = msg_k(y_s, scores, h2, t2, z32)
            fe = (fe3.reshape(_NSC, n_pad, 2, _SPLIT) * d_col[:, :, :, None]
                  ).reshape(_NSC, n_pad, 32)
            last_step = layer == _N_LAYERS - 1 and it == _N_ITERS - 1
            if not last_step:
                Fn = _norm_f(fe)
                A = A + att_k(Fn, Tn, h2, t2)
        ego_s = fe
        layer_outs.append(ego_s)
    all_s = _mean3(*layer_outs)
    all_emb = all_s.transpose(1, 0, 2).reshape(n_pad, _EMB)
    return all_emb[:n_users], all_emb[n_users:N]


# speedup vs baseline: 3.2944x; 1.0777x over previous
import functools

import jax
import jax.numpy as jnp
from jax import lax
from jax.experimental import pallas as pl
from jax.experimental.pallas import tpu as pltpu
from jax.experimental.pallas import tpu_sc as plsc

_EMB = 64
_NF = 4
_SPLIT = _EMB // _NF
_N_LAYERS = 2
_N_ITERS = 2

_NSC = 2
_NT = 16

_MSG_C = 512
_ATT_C = 512
_DEG_C = 1024


def _mds(start, size, mult):
    return pl.ds(pl.multiple_of(start, mult), size)


def _pad_to(x, n, axis=0):
    pad = [(0, 0)] * x.ndim
    pad[axis] = (0, n - x.shape[axis])
    return jnp.pad(x, pad)


def _mean3_body(a_ref, b_ref, c_ref, o_ref):
    o_ref[...] = (a_ref[...] + b_ref[...] + c_ref[...]) * (1.0 / 3.0)


def _mean3(a, b, c):
    n = a.shape[1]
    blk = n // 16
    return pl.pallas_call(
        _mean3_body,
        out_shape=jax.ShapeDtypeStruct(a.shape, a.dtype),
        grid=(_NSC, 16),
        in_specs=[pl.BlockSpec((1, blk, 32), lambda i, j: (i, j, 0))] * 3,
        out_specs=pl.BlockSpec((1, blk, 32), lambda i, j: (i, j, 0)),
    )(a, b, c)


def _norm_f(x):
    n = x.shape[1]
    xr = x.reshape(_NSC, n, 2, _SPLIT)
    nrm = jnp.sqrt(jnp.sum(xr * xr, axis=3, keepdims=True))
    return (xr / jnp.maximum(nrm, 1e-12)).reshape(_NSC, n, 32)


def _make_mesh():
    return plsc.VectorSubcoreMesh(core_axis_name="c", subcore_axis_name="s")


_SC_PARAMS = pltpu.CompilerParams(use_tc_tiling_on_sc=False,
                                  needs_layout_passes=False)


def _make_deg_kernel(n_pad, e_pad):
    rows_pt2 = n_pad * 2 // _NT
    edges_pt = e_pad // _NT
    chunks = edges_pt // _DEG_C
    fpc = _DEG_C * 2 // 128

    @functools.partial(
        pl.kernel,
        out_type=jax.ShapeDtypeStruct((_NSC, n_pad * 2), jnp.float32),
        mesh=_make_mesh(),
        compiler_params=_SC_PARAMS,
        scratch_types=[
            pltpu.VMEM_SHARED((n_pad * 2,), jnp.float32),
            pltpu.VMEM((fpc, 128), jnp.int32),
            pltpu.VMEM((fpc, 128), jnp.float32),
            pltpu.SemaphoreType.DMA,
        ],
    )
    def deg_kernel(sv, h2x2, z2f, dout, acc, idx2, vals, sem):
        c = lax.axis_index("c")
        s = lax.axis_index("s")
        sl = _mds(s * rows_pt2, rows_pt2, 8)
        pltpu.sync_copy(z2f.at[sl], acc.at[sl])
        plsc.subcore_barrier()
        base = s * edges_pt

        def chunk(i, carry):
            fb = (base + i * _DEG_C) // 64
            pltpu.sync_copy(h2x2.at[_mds(fb, fpc, 8)], idx2)
            pltpu.async_copy(sv.at[c].at[_mds(fb, fpc, 8)], vals, sem).wait()
            for j in range(fpc):
                pltpu.sync_copy(vals.at[j], acc.at[idx2.at[j]], add=True)
            return carry

        lax.fori_loop(0, chunks, chunk, 0)
        plsc.subcore_barrier()
        pltpu.sync_copy(acc.at[sl], dout.at[c, sl])

    return deg_kernel


def _make_msg_kernel(n_pad, e_pad):
    rows_pt = n_pad // _NT
    edges_pt = e_pad // _NT
    chunks = edges_pt // _MSG_C

    @functools.partial(
        pl.kernel,
        out_type=jax.ShapeDtypeStruct((_NSC, n_pad, 32), jnp.float32),
        mesh=_make_mesh(),
        compiler_params=_SC_PARAMS,
        scratch_types=[
            pltpu.VMEM_SHARED((n_pad, 32), jnp.float32),
            pltpu.VMEM((_MSG_C // 128, 128), jnp.int32),
            pltpu.VMEM((_MSG_C // 128, 128), jnp.int32),
            pltpu.VMEM((_MSG_C, 32), jnp.float32),
            pltpu.VMEM((_MSG_C * 2 // 128, 128), jnp.float32),
            pltpu.SemaphoreType.DMA,
        ],
    )
    def msg_kernel(y3, sv, h2, t2, z32, fe3,
                   acc, idx_h, idx_t, rows, sbuf, sem):
        c = lax.axis_index("c")
        s = lax.axis_index("s")
        sl = _mds(s * rows_pt, rows_pt, 8)
        pltpu.sync_copy(z32.at[sl], acc.at[sl])
        plsc.subcore_barrier()
        base = s * edges_pt

        def chunk(i, carry):
            eb = base + i * _MSG_C
            rb = eb // 128
            pltpu.sync_copy(t2.at[_mds(rb, _MSG_C // 128, 4)], idx_t)
            pltpu.sync_copy(h2.at[_mds(rb, _MSG_C // 128, 4)], idx_h)
            cps = [pltpu.async_copy(
                sv.at[c].at[_mds(eb // 64, _MSG_C * 2 // 128, 8)], sbuf, sem)]
            for j in range(_MSG_C // 128):
                cps.append(pltpu.async_copy(
                    y3.at[c].at[idx_t.at[j]],
                    rows.at[pl.ds(j * 128, 128)], sem))
            for cp in cps:
                cp.wait()

            @plsc.parallel_loop(0, _MSG_C // 8, 1, unroll=4)
            def mulgrp(g):
                v = sbuf[g // 8, _mds((g % 8) * 16, 16, 16)]
                for k in range(8):
                    r = g * 8 + k
                    rows[r, pl.ds(0, 16)] = rows[r, pl.ds(0, 16)] * v[2 * k]
                    rows[r, pl.ds(16, 16)] = rows[r, pl.ds(16, 16)] * v[2 * k + 1]
            for j in range(_MSG_C // 128):
                pltpu.sync_copy(rows.at[pl.ds(j * 128, 128)],
                                acc.at[idx_h.at[j]], add=True)
            return carry

        lax.fori_loop(0, chunks, chunk, 0)
        plsc.subcore_barrier()
        pltpu.sync_copy(acc.at[sl], fe3.at[c, sl])

    return msg_kernel


def _make_att_kernel(n_pad, e_pad):
    edges_pt = e_pad // _NT
    chunks = edges_pt // _ATT_C
    fpc = _ATT_C * 2 // 128

    @functools.partial(
        pl.kernel,
        out_type=jax.ShapeDtypeStruct((_NSC, e_pad * 2 // 128, 128),
                                      jnp.float32),
        mesh=_make_mesh(),
        compiler_params=_SC_PARAMS,
        scratch_types=[
            pltpu.VMEM((_ATT_C // 128, 128), jnp.int32),
            pltpu.VMEM((_ATT_C // 128, 128), jnp.int32),
            pltpu.VMEM((_ATT_C, 32), jnp.float32),
            pltpu.VMEM((_ATT_C, 32), jnp.float32),
            pltpu.VMEM((_ATT_C // 128, 128), jnp.int32),
            pltpu.VMEM((_ATT_C // 128, 128), jnp.int32),
            pltpu.VMEM((_ATT_C, 32), jnp.float32),
            pltpu.VMEM((_ATT_C, 32), jnp.float32),
            pltpu.VMEM((16, fpc, 128), jnp.float32),
            pltpu.VMEM((fpc, 128), jnp.float32),
            pltpu.SemaphoreType.DMA,
            pltpu.SemaphoreType.DMA,
        ],
    )
    def att_kernel(fn, tn, h2, t2, out, ih0, it0, ra0, rb0,
                   ih1, it1, ra1, rb1, pt, psum, sem0, sem1):
        c = lax.axis_index("c")
        s = lax.axis_index("s")
        base = s * edges_pt
        kio = lax.iota(jnp.int32, 16)

        def issue(ci, ih, it, ra, rb, sem):
            eb = base + ci * _ATT_C
            rbase = eb // 128
            pltpu.sync_copy(h2.at[_mds(rbase, _ATT_C // 128, 4)], ih)
            pltpu.sync_copy(t2.at[_mds(rbase, _ATT_C // 128, 4)], it)
            for j in range(_ATT_C // 128):
                pltpu.async_copy(fn.at[c].at[ih.at[j]],
                                 ra.at[pl.ds(j * 128, 128)], sem)
                pltpu.async_copy(tn.at[c].at[it.at[j]],
                                 rb.at[pl.ds(j * 128, 128)], sem)

        def drain(ra, rb, sem):
            pltpu.make_async_copy(fn.at[c].at[pl.ds(0, _ATT_C)], ra,
                                  sem).wait()
            pltpu.make_async_copy(tn.at[c].at[pl.ds(0, _ATT_C)], rb,
                                  sem).wait()

        def compute_write(ci, ra, rb):
            eb = base + ci * _ATT_C

            @plsc.parallel_loop(0, _ATT_C, 1, unroll=8)
            def mulrow(r):
                jj = r // 64
                col = (r % 64) * 2
                v0 = ra[r, pl.ds(0, 16)] * rb[r, pl.ds(0, 16)]
                v1 = ra[r, pl.ds(16, 16)] * rb[r, pl.ds(16, 16)]
                jv = jnp.full((16,), jj, jnp.int32)
                cv = jnp.full((16,), col, jnp.int32)
                plsc.store_scatter(pt, [kio, jv, cv], v0)
                plsc.store_scatter(pt, [kio, jv, cv + 1], v1)

            @plsc.parallel_loop(0, fpc * 8, 1, unroll=2)
            def redgrp(g):
                jj = g // 8
                col = (g % 8) * 16
                acc16 = pt[0, jj, _mds(col, 16, 16)]
                for k in range(1, 16):
                    acc16 = acc16 + pt[k, jj, _mds(col, 16, 16)]
                psum[jj, _mds(col, 16, 16)] = acc16

            pltpu.sync_copy(psum, out.at[c].at[_mds(eb // 64, fpc, 8)])

        issue(0, ih0, it0, ra0, rb0, sem0)

        def pair(g, carry):
            drain(ra0, rb0, sem0)
            issue(2 * g + 1, ih1, it1, ra1, rb1, sem1)
            compute_write(2 * g, ra0, rb0)
            drain(ra1, rb1, sem1)

            @pl.when(g + 1 < chunks // 2)
            def _():
                issue(2 * g + 2, ih0, it0, ra0, rb0, sem0)

            compute_write(2 * g + 1, ra1, rb1)
            return carry

        lax.fori_loop(0, chunks // 2, pair, 0)

    return att_kernel


def kernel(user_embedding, item_embedding, all_h, all_t):
    n_users = user_embedding.shape[0]
    N = n_users + item_embedding.shape[0]
    E = all_h.shape[0]
    n_pad = ((N + 1 + _NT * 64 - 1) // (_NT * 64)) * (_NT * 64)
    epc = _NT * _DEG_C
    e_pad = ((E + epc - 1) // epc) * epc
    er = e_pad * 2 // 128

    deg_k = _make_deg_kernel(n_pad, e_pad)
    msg_k = _make_msg_kernel(n_pad, e_pad)
    att_k = _make_att_kernel(n_pad, e_pad)

    ego = jnp.concatenate([user_embedding, item_embedding], axis=0)
    ego = _pad_to(ego, n_pad)
    ego_s = ego.reshape(n_pad, _NSC, 32).transpose(1, 0, 2)
    h_p = _pad_to(all_h, e_pad).at[E:].set(N)
    t_p = _pad_to(all_t, e_pad).at[E:].set(N)
    h2 = h_p.reshape(e_pad // 128, 128)
    t2 = t_p.reshape(e_pad // 128, 128)
    h2x2 = (jnp.repeat(h_p.reshape(er, 64) * 2, 2, axis=1)
            + (jnp.arange(128, dtype=jnp.int32) % 2)[None, :])
    z2f = jnp.zeros((n_pad * 2,), jnp.float32)
    z32 = jnp.zeros((n_pad, 32), jnp.float32)

    A = jnp.ones((_NSC, er, 128), dtype=jnp.float32)
    lane_even = (jnp.arange(128) % 2 == 0)
    layer_outs = [ego_s]
    for layer in range(_N_LAYERS):
        Tn = jnp.tanh(_norm_f(ego_s))
        ego_r = ego_s.reshape(_NSC, n_pad, 2, _SPLIT)
        fe = None
        for it in range(_N_ITERS):
            ex = jnp.exp(A)
            den = (ex[0] + ex[1])
            den = den[:, 0::2] + den[:, 1::2]
            scores = ex / jnp.repeat(den, 2, axis=1)[None]
            dflat = deg_k(scores, h2x2, z2f)
            d_col = lax.rsqrt(jnp.maximum(dflat, 1e-30)).reshape(
                _NSC, n_pad, 2)
            y_s = (ego_r * d_col[:, :, :, None]).reshape(_NSC, n_pad, 32)
            fe3 = msg_k(y_s, scores, h2, t2, z32)
            fe = (fe3.reshape(_NSC, n_pad, 2, _SPLIT) * d_col[:, :, :, None]
                  ).reshape(_NSC, n_pad, 32)
            last_step = layer == _N_LAYERS - 1 and it == _N_ITERS - 1
            if not last_step:
                Fn = _norm_f(fe)
                A = A + att_k(Fn, Tn, h2, t2)
        ego_s = fe
        layer_outs.append(ego_s)
    all_s = _mean3(*layer_outs)
    all_emb = all_s.transpose(1, 0, 2).reshape(n_pad, _EMB)
    return all_emb[:n_users], all_emb[n_users:N]


# speedup vs baseline: 3.3669x; 1.0220x over previous
import functools

import jax
import jax.numpy as jnp
from jax import lax
from jax.experimental import pallas as pl
from jax.experimental.pallas import tpu as pltpu
from jax.experimental.pallas import tpu_sc as plsc

_EMB = 64
_NF = 4
_SPLIT = _EMB // _NF
_N_LAYERS = 2
_N_ITERS = 2

_NSC = 2
_NT = 16

_MSG_C = 512
_ATT_C = 512
_DEG_C = 1024


def _mds(start, size, mult):
    return pl.ds(pl.multiple_of(start, mult), size)


def _pad_to(x, n, axis=0):
    pad = [(0, 0)] * x.ndim
    pad[axis] = (0, n - x.shape[axis])
    return jnp.pad(x, pad)


def _mean3_body(a_ref, b_ref, c_ref, o_ref):
    o_ref[...] = (a_ref[...] + b_ref[...] + c_ref[...]) * (1.0 / 3.0)


def _mean3(a, b, c):
    n = a.shape[1]
    blk = n // 16
    return pl.pallas_call(
        _mean3_body,
        out_shape=jax.ShapeDtypeStruct(a.shape, a.dtype),
        grid=(_NSC, 16),
        in_specs=[pl.BlockSpec((1, blk, 32), lambda i, j: (i, j, 0))] * 3,
        out_specs=pl.BlockSpec((1, blk, 32), lambda i, j: (i, j, 0)),
    )(a, b, c)


def _norm_f(x):
    n = x.shape[1]
    xr = x.reshape(_NSC, n, 2, _SPLIT)
    nrm = jnp.sqrt(jnp.sum(xr * xr, axis=3, keepdims=True))
    return (xr / jnp.maximum(nrm, 1e-12)).reshape(_NSC, n, 32)


def _make_mesh():
    return plsc.VectorSubcoreMesh(core_axis_name="c", subcore_axis_name="s")


_SC_PARAMS = pltpu.CompilerParams(use_tc_tiling_on_sc=False,
                                  needs_layout_passes=False)


def _make_deg_kernel(n_pad, e_pad):
    rows_pt2 = n_pad * 2 // _NT
    edges_pt = e_pad // _NT
    chunks = edges_pt // _DEG_C
    fpc = _DEG_C * 2 // 128

    @functools.partial(
        pl.kernel,
        out_type=jax.ShapeDtypeStruct((_NSC, n_pad * 2), jnp.float32),
        mesh=_make_mesh(),
        compiler_params=_SC_PARAMS,
        scratch_types=[
            pltpu.VMEM_SHARED((n_pad * 2,), jnp.float32),
            pltpu.VMEM((fpc, 128), jnp.int32),
            pltpu.VMEM((fpc, 128), jnp.float32),
            pltpu.SemaphoreType.DMA,
        ],
    )
    def deg_kernel(sv, h2x2, z2f, dout, acc, idx2, vals, sem):
        c = lax.axis_index("c")
        s = lax.axis_index("s")
        sl = _mds(s * rows_pt2, rows_pt2, 8)
        pltpu.sync_copy(z2f.at[sl], acc.at[sl])
        plsc.subcore_barrier()
        base = s * edges_pt

        def chunk(i, carry):
            fb = (base + i * _DEG_C) // 64
            pltpu.sync_copy(h2x2.at[_mds(fb, fpc, 8)], idx2)
            pltpu.async_copy(sv.at[c].at[_mds(fb, fpc, 8)], vals, sem).wait()
            for j in range(fpc):
                pltpu.sync_copy(vals.at[j], acc.at[idx2.at[j]], add=True)
            return carry

        lax.fori_loop(0, chunks, chunk, 0)
        plsc.subcore_barrier()
        pltpu.sync_copy(acc.at[sl], dout.at[c, sl])

    return deg_kernel


def _make_msg_kernel(n_pad, e_pad):
    rows_pt = n_pad // _NT
    edges_pt = e_pad // _NT
    C = 256
    chunks = edges_pt // C

    @functools.partial(
        pl.kernel,
        out_type=jax.ShapeDtypeStruct((_NSC, n_pad, 32), jnp.float32),
        mesh=_make_mesh(),
        compiler_params=_SC_PARAMS,
        scratch_types=[
            pltpu.VMEM_SHARED((n_pad, 32), jnp.float32),
            pltpu.VMEM((C // 128, 128), jnp.int32),
            pltpu.VMEM((C // 128, 128), jnp.int32),
            pltpu.VMEM((C, 32), jnp.float32),
            pltpu.VMEM((C * 2 // 128, 128), jnp.float32),
            pltpu.VMEM((C // 128, 128), jnp.int32),
            pltpu.VMEM((C // 128, 128), jnp.int32),
            pltpu.VMEM((C, 32), jnp.float32),
            pltpu.VMEM((C * 2 // 128, 128), jnp.float32),
            pltpu.SemaphoreType.DMA,
            pltpu.SemaphoreType.DMA,
        ],
    )
    def msg_kernel(y3, sv, h2, t2, z32, fe3,
                   acc, ih0, it0, rows0, sb0, ih1, it1, rows1, sb1,
                   sem0, sem1):
        c = lax.axis_index("c")
        s = lax.axis_index("s")
        sl = _mds(s * rows_pt, rows_pt, 8)
        pltpu.sync_copy(z32.at[sl], acc.at[sl])
        plsc.subcore_barrier()
        base = s * edges_pt

        def issue(ci, ih, it, rows, sb, sem):
            eb = base + ci * C
            rb = eb // 128
            pltpu.sync_copy(t2.at[_mds(rb, C // 128, 2)], it)
            pltpu.sync_copy(h2.at[_mds(rb, C // 128, 2)], ih)
            pltpu.async_copy(sv.at[c].at[_mds(eb // 64, C * 2 // 128, 4)],
                             sb, sem)
            for j in range(C // 128):
                pltpu.async_copy(y3.at[c].at[it.at[j]],
                                 rows.at[pl.ds(j * 128, 128)], sem)

        def drain(rows, sb, sem):
            pltpu.make_async_copy(y3.at[c].at[pl.ds(0, C)], rows, sem).wait()
            pltpu.make_async_copy(sv.at[c].at[pl.ds(0, C * 2 // 128)], sb,
                                  sem).wait()

        def compute_scatter(ih, rows, sb):
            @plsc.parallel_loop(0, C // 8, 1, unroll=4)
            def mulgrp(g):
                v = sb[g // 8, _mds((g % 8) * 16, 16, 16)]
                for k in range(8):
                    r = g * 8 + k
                    rows[r, pl.ds(0, 16)] = rows[r, pl.ds(0, 16)] * v[2 * k]
                    rows[r, pl.ds(16, 16)] = (rows[r, pl.ds(16, 16)]
                                              * v[2 * k + 1])

            for j in range(C // 128):
                pltpu.sync_copy(rows.at[pl.ds(j * 128, 128)],
                                acc.at[ih.at[j]], add=True)

        issue(0, ih0, it0, rows0, sb0, sem0)

        def pair(g, carry):
            issue(2 * g + 1, ih1, it1, rows1, sb1, sem1)
            drain(rows0, sb0, sem0)
            compute_scatter(ih0, rows0, sb0)

            @pl.when(g + 1 < chunks // 2)
            def _():
                issue(2 * g + 2, ih0, it0, rows0, sb0, sem0)

            drain(rows1, sb1, sem1)
            compute_scatter(ih1, rows1, sb1)
            return carry

        lax.fori_loop(0, chunks // 2, pair, 0)
        plsc.subcore_barrier()
        pltpu.sync_copy(acc.at[sl], fe3.at[c, sl])

    return msg_kernel


def _make_att_kernel(n_pad, e_pad):
    edges_pt = e_pad // _NT
    chunks = edges_pt // _ATT_C
    fpc = _ATT_C * 2 // 128

    @functools.partial(
        pl.kernel,
        out_type=jax.ShapeDtypeStruct((_NSC, e_pad * 2 // 128, 128),
                                      jnp.float32),
        mesh=_make_mesh(),
        compiler_params=_SC_PARAMS,
        scratch_types=[
            pltpu.VMEM((_ATT_C // 128, 128), jnp.int32),
            pltpu.VMEM((_ATT_C // 128, 128), jnp.int32),
            pltpu.VMEM((_ATT_C, 32), jnp.float32),
            pltpu.VMEM((_ATT_C, 32), jnp.float32),
            pltpu.VMEM((_ATT_C // 128, 128), jnp.int32),
            pltpu.VMEM((_ATT_C // 128, 128), jnp.int32),
            pltpu.VMEM((_ATT_C, 32), jnp.float32),
            pltpu.VMEM((_ATT_C, 32), jnp.float32),
            pltpu.VMEM((16, fpc, 128), jnp.float32),
            pltpu.VMEM((fpc, 128), jnp.float32),
            pltpu.SemaphoreType.DMA,
            pltpu.SemaphoreType.DMA,
        ],
    )
    def att_kernel(fn, tn, h2, t2, out, ih0, it0, ra0, rb0,
                   ih1, it1, ra1, rb1, pt, psum, sem0, sem1):
        c = lax.axis_index("c")
        s = lax.axis_index("s")
        base = s * edges_pt
        kio = lax.iota(jnp.int32, 16)

        def issue(ci, ih, it, ra, rb, sem):
            eb = base + ci * _ATT_C
            rbase = eb // 128
            pltpu.sync_copy(h2.at[_mds(rbase, _ATT_C // 128, 4)], ih)
            pltpu.sync_copy(t2.at[_mds(rbase, _ATT_C // 128, 4)], it)
            for j in range(_ATT_C // 128):
                pltpu.async_copy(fn.at[c].at[ih.at[j]],
                                 ra.at[pl.ds(j * 128, 128)], sem)
                pltpu.async_copy(tn.at[c].at[it.at[j]],
                                 rb.at[pl.ds(j * 128, 128)], sem)

        def drain(ra, rb, sem):
            pltpu.make_async_copy(fn.at[c].at[pl.ds(0, _ATT_C)], ra,
                                  sem).wait()
            pltpu.make_async_copy(tn.at[c].at[pl.ds(0, _ATT_C)], rb,
                                  sem).wait()

        def compute_write(ci, ra, rb):
            eb = base + ci * _ATT_C

            @plsc.parallel_loop(0, _ATT_C, 1, unroll=8)
            def mulrow(r):
                jj = r // 64
                col = (r % 64) * 2
                v0 = ra[r, pl.ds(0, 16)] * rb[r, pl.ds(0, 16)]
                v1 = ra[r, pl.ds(16, 16)] * rb[r, pl.ds(16, 16)]
                jv = jnp.full((16,), jj, jnp.int32)
                cv = jnp.full((16,), col, jnp.int32)
                plsc.store_scatter(pt, [kio, jv, cv], v0)
                plsc.store_scatter(pt, [kio, jv, cv + 1], v1)

            @plsc.parallel_loop(0, fpc * 8, 1, unroll=2)
            def redgrp(g):
                jj = g // 8
                col = (g % 8) * 16
                acc16 = pt[0, jj, _mds(col, 16, 16)]
                for k in range(1, 16):
                    acc16 = acc16 + pt[k, jj, _mds(col, 16, 16)]
                psum[jj, _mds(col, 16, 16)] = acc16

            pltpu.sync_copy(psum, out.at[c].at[_mds(eb // 64, fpc, 8)])

        issue(0, ih0, it0, ra0, rb0, sem0)

        def pair(g, carry):
            drain(ra0, rb0, sem0)
            issue(2 * g + 1, ih1, it1, ra1, rb1, sem1)
            compute_write(2 * g, ra0, rb0)
            drain(ra1, rb1, sem1)

            @pl.when(g + 1 < chunks // 2)
            def _():
                issue(2 * g + 2, ih0, it0, ra0, rb0, sem0)

            compute_write(2 * g + 1, ra1, rb1)
            return carry

        lax.fori_loop(0, chunks // 2, pair, 0)

    return att_kernel


def kernel(user_embedding, item_embedding, all_h, all_t):
    n_users = user_embedding.shape[0]
    N = n_users + item_embedding.shape[0]
    E = all_h.shape[0]
    n_pad = ((N + 1 + _NT * 64 - 1) // (_NT * 64)) * (_NT * 64)
    epc = _NT * _DEG_C
    e_pad = ((E + epc - 1) // epc) * epc
    er = e_pad * 2 // 128

    deg_k = _make_deg_kernel(n_pad, e_pad)
    msg_k = _make_msg_kernel(n_pad, e_pad)
    att_k = _make_att_kernel(n_pad, e_pad)

    ego = jnp.concatenate([user_embedding, item_embedding], axis=0)
    ego = _pad_to(ego, n_pad)
    ego_s = ego.reshape(n_pad, _NSC, 32).transpose(1, 0, 2)
    h_p = _pad_to(all_h, e_pad).at[E:].set(N)
    t_p = _pad_to(all_t, e_pad).at[E:].set(N)
    h2 = h_p.reshape(e_pad // 128, 128)
    t2 = t_p.reshape(e_pad // 128, 128)
    h2x2 = (jnp.repeat(h_p.reshape(er, 64) * 2, 2, axis=1)
            + (jnp.arange(128, dtype=jnp.int32) % 2)[None, :])
    z2f = jnp.zeros((n_pad * 2,), jnp.float32)
    z32 = jnp.zeros((n_pad, 32), jnp.float32)

    A = jnp.ones((_NSC, er, 128), dtype=jnp.float32)
    lane_even = (jnp.arange(128) % 2 == 0)
    layer_outs = [ego_s]
    for layer in range(_N_LAYERS):
        Tn = jnp.tanh(_norm_f(ego_s))
        ego_r = ego_s.reshape(_NSC, n_pad, 2, _SPLIT)
        fe = None
        for it in range(_N_ITERS):
            ex = jnp.exp(A)
            den = (ex[0] + ex[1])
            den = den[:, 0::2] + den[:, 1::2]
            scores = ex / jnp.repeat(den, 2, axis=1)[None]
            dflat = deg_k(scores, h2x2, z2f)
            d_col = lax.rsqrt(jnp.maximum(dflat, 1e-30)).reshape(
                _NSC, n_pad, 2)
            y_s = (ego_r * d_col[:, :, :, None]).reshape(_NSC, n_pad, 32)
            fe3 = msg_k(y_s, scores, h2, t2, z32)
            fe = (fe3.reshape(_NSC, n_pad, 2, _SPLIT) * d_col[:, :, :, None]
                  ).reshape(_NSC, n_pad, 32)
            last_step = layer == _N_LAYERS - 1 and it == _N_ITERS - 1
            if not last_step:
                Fn = _norm_f(fe)
                A = A + att_k(Fn, Tn, h2, t2)
        ego_s = fe
        layer_outs.append(ego_s)
    all_s = _mean3(*layer_outs)
    all_emb = all_s.transpose(1, 0, 2).reshape(n_pad, _EMB)
    return all_emb[:n_users], all_emb[n_users:N]
